# speedup vs baseline: 1.4801x; 1.4801x over previous
import jax
import jax.numpy as jnp
from jax.experimental import pallas as pl
from jax.experimental.pallas import tpu as pltpu

N = 4096
D_IN = 128
HEADS = 4
HID = 16
NC_OUT = 16
NG = 64
EPS = 0.0001

BI = 256
NB = N // BI

_HIGH = jax.lax.Precision.HIGHEST


def _proj_body(x_ref, wf_ref, pa_ref, pb_ref, h_ref, sa_ref, sb_ref):
    h = jax.lax.dot(x_ref[...], wf_ref[...], precision=_HIGH)
    h_ref[...] = h
    sa_ref[...] = jax.lax.dot(h, pa_ref[...], precision=_HIGH)
    sb_ref[...] = jax.lax.dot(h, pb_ref[...], precision=_HIGH)


def _main_body(mask_ref, h_ref, sa_ref, sbt_ref, bat_ref,
               w1_ref, b1_ref, w2_ref, b2_ref,
               logyp_ref, preds_ref, sums_ref, cnt_ref):
    ib = pl.program_id(0)

    @pl.when(ib == 0)
    def _init():
        sums_ref[...] = jnp.zeros_like(sums_ref)
        cnt_ref[...] = jnp.zeros_like(cnt_ref)

    edge = (mask_ref[...] == 1).astype(jnp.float32)
    aggs = []
    for hd in range(HEADS):
        t = sa_ref[:, hd:hd + 1] + sbt_ref[hd:hd + 1, :]
        l = jnp.maximum(t, 0.01 * t)
        w = 1.0 / (1.0 + jnp.exp(-l))
        wm = w * edge
        hh = h_ref[:, hd * HID:(hd + 1) * HID]
        aggs.append(jnp.dot(wm, hh, preferred_element_type=jnp.float32))
    agg = jnp.concatenate(aggs, axis=1)

    z = jax.lax.dot(agg, w1_ref[...], precision=_HIGH) + b1_ref[...]
    z = jnp.maximum(z, 0.01 * z)
    z = jax.lax.dot(z, w2_ref[...], precision=_HIGH) + b2_ref[...]
    tmp = jnp.exp(z - jnp.max(z, axis=-1, keepdims=True)) + EPS
    preds = tmp / jnp.sum(tmp, axis=-1, keepdims=True)
    preds_ref[...] = preds

    bat = bat_ref[0]
    gi = jax.lax.broadcasted_iota(jnp.int32, (NG, BI), 0)
    oh = (gi == bat).astype(jnp.float32)
    sums_ref[...] += jax.lax.dot(oh, preds, precision=_HIGH)
    cnt_ref[...] += jnp.sum(oh, axis=1, keepdims=True)

    @pl.when(ib == NB - 1)
    def _fin():
        yp = sums_ref[...] / jnp.maximum(cnt_ref[...], 1.0)
        logyp_ref[...] = jnp.log(yp)


@jax.jit
def kernel(x, batch, mask, Wf, W1, b1, W2, b2, phi):
    phi_a = phi[:, :HID, 0]
    phi_b = phi[:, HID:, 0]
    eye = jnp.eye(HEADS, dtype=jnp.float32)
    pa = (phi_a[:, :, None] * eye[:, None, :]).reshape(HEADS * HID, HEADS)
    pb = (phi_b[:, :, None] * eye[:, None, :]).reshape(HEADS * HID, HEADS)

    h, sa, sb = pl.pallas_call(
        _proj_body,
        out_shape=[
            jax.ShapeDtypeStruct((N, HEADS * HID), jnp.float32),
            jax.ShapeDtypeStruct((N, HEADS), jnp.float32),
            jax.ShapeDtypeStruct((N, HEADS), jnp.float32),
        ],
    )(x, Wf, pa, pb)

    sbt = sb.T
    bat3 = batch.reshape(NB, 1, BI)
    b1r = b1.reshape(1, HID)
    b2r = b2.reshape(1, NC_OUT)

    grid = (NB,)
    logyp, preds = pl.pallas_call(
        _main_body,
        grid=grid,
        in_specs=[
            pl.BlockSpec((BI, N), lambda i: (i, 0)),
            pl.BlockSpec((N, HEADS * HID), lambda i: (0, 0)),
            pl.BlockSpec((BI, HEADS), lambda i: (i, 0)),
            pl.BlockSpec((HEADS, N), lambda i: (0, 0)),
            pl.BlockSpec((1, 1, BI), lambda i: (i, 0, 0)),
            pl.BlockSpec((HEADS * HID, HID), lambda i: (0, 0)),
            pl.BlockSpec((1, HID), lambda i: (0, 0)),
            pl.BlockSpec((HID, NC_OUT), lambda i: (0, 0)),
            pl.BlockSpec((1, NC_OUT), lambda i: (0, 0)),
        ],
        out_specs=[
            pl.BlockSpec((NG, NC_OUT), lambda i: (0, 0)),
            pl.BlockSpec((BI, NC_OUT), lambda i: (i, 0)),
        ],
        out_shape=[
            jax.ShapeDtypeStruct((NG, NC_OUT), jnp.float32),
            jax.ShapeDtypeStruct((N, NC_OUT), jnp.float32),
        ],
        scratch_shapes=[
            pltpu.VMEM((NG, NC_OUT), jnp.float32),
            pltpu.VMEM((NG, 1), jnp.float32),
        ],
        compiler_params=pltpu.CompilerParams(
            dimension_semantics=("arbitrary",),
        ),
    )(mask, h, sa, sbt, bat3, W1, b1r, W2, b2r)

    return (logyp, preds)


# speedup vs baseline: 1.8321x; 1.2378x over previous
import jax
import jax.numpy as jnp
from jax.experimental import pallas as pl
from jax.experimental.pallas import tpu as pltpu

N = 4096
D_IN = 128
HEADS = 4
HID = 16
NC_OUT = 16
NG = 64
EPS = 0.0001

BI = 256
NB = N // BI

_HIGH = jax.lax.Precision.HIGHEST


def _proj_body(x_ref, wf_ref, pa_ref, pb_ref, h_ref, sa_ref, sb_ref):
    h = jax.lax.dot(x_ref[...], wf_ref[...], precision=_HIGH)
    h_ref[...] = h
    sa_ref[...] = jax.lax.dot(h, pa_ref[...], precision=_HIGH)
    sb_ref[...] = jax.lax.dot(h, pb_ref[...], precision=_HIGH)


def _main_body(mask_ref, h_ref, sa_ref, sbt_ref, bat_ref,
               w1_ref, b1_ref, w2_ref, b2_ref,
               logyp_ref, preds_ref, sums_ref, cnt_ref):
    ib = pl.program_id(0)

    @pl.when(ib == 0)
    def _init():
        sums_ref[...] = jnp.zeros_like(sums_ref)
        cnt_ref[...] = jnp.zeros_like(cnt_ref)

    e2 = jnp.where(mask_ref[...] == 1, 0.5, 0.0)
    aggs = []
    for hd in range(HEADS):
        t = sa_ref[:, hd:hd + 1] + sbt_ref[hd:hd + 1, :]
        m = jnp.maximum(t, 0.01 * t)
        v = jnp.tanh(m)
        wm = e2 + e2 * v
        hh = h_ref[:, hd * HID:(hd + 1) * HID]
        aggs.append(jnp.dot(wm, hh, preferred_element_type=jnp.float32))
    agg = jnp.concatenate(aggs, axis=1)

    z = jax.lax.dot(agg, w1_ref[...], precision=_HIGH) + b1_ref[...]
    z = jnp.maximum(z, 0.01 * z)
    z = jax.lax.dot(z, w2_ref[...], precision=_HIGH) + b2_ref[...]
    tmp = jnp.exp(z - jnp.max(z, axis=-1, keepdims=True)) + EPS
    preds = tmp / jnp.sum(tmp, axis=-1, keepdims=True)
    preds_ref[...] = preds

    bat = bat_ref[0]
    gi = jax.lax.broadcasted_iota(jnp.int32, (NG, BI), 0)
    oh = (gi == bat).astype(jnp.float32)
    sums_ref[...] += jax.lax.dot(oh, preds, precision=_HIGH)
    cnt_ref[...] += jnp.sum(oh, axis=1, keepdims=True)

    @pl.when(ib == NB - 1)
    def _fin():
        yp = sums_ref[...] / jnp.maximum(cnt_ref[...], 1.0)
        logyp_ref[...] = jnp.log(yp)


@jax.jit
def kernel(x, batch, mask, Wf, W1, b1, W2, b2, phi):
    phi_a = phi[:, :HID, 0]
    phi_b = phi[:, HID:, 0]
    eye = jnp.eye(HEADS, dtype=jnp.float32)
    pa = 0.5 * (phi_a[:, :, None] * eye[:, None, :]).reshape(HEADS * HID, HEADS)
    pb = 0.5 * (phi_b[:, :, None] * eye[:, None, :]).reshape(HEADS * HID, HEADS)

    h, sa, sb = pl.pallas_call(
        _proj_body,
        out_shape=[
            jax.ShapeDtypeStruct((N, HEADS * HID), jnp.float32),
            jax.ShapeDtypeStruct((N, HEADS), jnp.float32),
            jax.ShapeDtypeStruct((N, HEADS), jnp.float32),
        ],
    )(x, Wf, pa, pb)

    sbt = sb.T
    bat3 = batch.reshape(NB, 1, BI)
    b1r = b1.reshape(1, HID)
    b2r = b2.reshape(1, NC_OUT)

    grid = (NB,)
    logyp, preds = pl.pallas_call(
        _main_body,
        grid=grid,
        in_specs=[
            pl.BlockSpec((BI, N), lambda i: (i, 0)),
            pl.BlockSpec((N, HEADS * HID), lambda i: (0, 0)),
            pl.BlockSpec((BI, HEADS), lambda i: (i, 0)),
            pl.BlockSpec((HEADS, N), lambda i: (0, 0)),
            pl.BlockSpec((1, 1, BI), lambda i: (i, 0, 0)),
            pl.BlockSpec((HEADS * HID, HID), lambda i: (0, 0)),
            pl.BlockSpec((1, HID), lambda i: (0, 0)),
            pl.BlockSpec((HID, NC_OUT), lambda i: (0, 0)),
            pl.BlockSpec((1, NC_OUT), lambda i: (0, 0)),
        ],
        out_specs=[
            pl.BlockSpec((NG, NC_OUT), lambda i: (0, 0)),
            pl.BlockSpec((BI, NC_OUT), lambda i: (i, 0)),
        ],
        out_shape=[
            jax.ShapeDtypeStruct((NG, NC_OUT), jnp.float32),
            jax.ShapeDtypeStruct((N, NC_OUT), jnp.float32),
        ],
        scratch_shapes=[
            pltpu.VMEM((NG, NC_OUT), jnp.float32),
            pltpu.VMEM((NG, 1), jnp.float32),
        ],
        compiler_params=pltpu.CompilerParams(
            dimension_semantics=("arbitrary",),
        ),
    )(mask, h, sa, sbt, bat3, W1, b1r, W2, b2r)

    return (logyp, preds)


# speedup vs baseline: 2.0894x; 1.1404x over previous
import jax
import jax.numpy as jnp
from jax.experimental import pallas as pl
from jax.experimental.pallas import tpu as pltpu

N = 4096
D_IN = 128
HEADS = 4
HID = 16
NC_OUT = 16
NG = 64
EPS = 0.0001

BI = 256
NB = N // BI

_HIGH = jax.lax.Precision.HIGHEST


def _proj_body(x_ref, wf_ref, pa_ref, pb_ref, h_ref, sa_ref, sb_ref):
    h = jax.lax.dot(x_ref[...], wf_ref[...], precision=_HIGH)
    h_ref[...] = h
    sa_ref[...] = jax.lax.dot(h, pa_ref[...], precision=_HIGH)
    sb_ref[...] = jax.lax.dot(h, pb_ref[...], precision=_HIGH)


def _main_body(mask_ref, h_ref, sa_ref, sbt_ref, bat_ref,
               w1_ref, b1_ref, w2_ref, b2_ref,
               logyp_ref, preds_ref, sums_ref, cnt_ref):
    ib = pl.program_id(0)

    @pl.when(ib == 0)
    def _init():
        sums_ref[...] = jnp.zeros_like(sums_ref)
        cnt_ref[...] = jnp.zeros_like(cnt_ref)

    bf = jnp.bfloat16
    e2 = jnp.where(mask_ref[...] == 1, 0.5, 0.0).astype(bf)
    h_bf = h_ref[...].astype(bf)
    sa_bf = sa_ref[...].astype(bf)
    sbt_bf = sbt_ref[...].astype(bf)
    base = jnp.dot(e2, h_bf, preferred_element_type=jnp.float32)
    aggs = []
    for hd in range(HEADS):
        t = sa_bf[:, hd:hd + 1] + sbt_bf[hd:hd + 1, :]
        m = jnp.maximum(t, bf(0.01) * t)
        v = jnp.tanh(m)
        wmv = e2 * v
        hh = h_bf[:, hd * HID:(hd + 1) * HID]
        aggs.append(jnp.dot(wmv, hh, preferred_element_type=jnp.float32))
    agg = base + jnp.concatenate(aggs, axis=1)

    z = jax.lax.dot(agg, w1_ref[...], precision=_HIGH) + b1_ref[...]
    z = jnp.maximum(z, 0.01 * z)
    z = jax.lax.dot(z, w2_ref[...], precision=_HIGH) + b2_ref[...]
    tmp = jnp.exp(z - jnp.max(z, axis=-1, keepdims=True)) + EPS
    preds = tmp / jnp.sum(tmp, axis=-1, keepdims=True)
    preds_ref[...] = preds

    bat = bat_ref[0]
    gi = jax.lax.broadcasted_iota(jnp.int32, (NG, BI), 0)
    oh = (gi == bat).astype(jnp.float32)
    sums_ref[...] += jax.lax.dot(oh, preds, precision=_HIGH)
    cnt_ref[...] += jnp.sum(oh, axis=1, keepdims=True)

    @pl.when(ib == NB - 1)
    def _fin():
        yp = sums_ref[...] / jnp.maximum(cnt_ref[...], 1.0)
        logyp_ref[...] = jnp.log(yp)


@jax.jit
def kernel(x, batch, mask, Wf, W1, b1, W2, b2, phi):
    phi_a = phi[:, :HID, 0]
    phi_b = phi[:, HID:, 0]
    eye = jnp.eye(HEADS, dtype=jnp.float32)
    pa = 0.5 * (phi_a[:, :, None] * eye[:, None, :]).reshape(HEADS * HID, HEADS)
    pb = 0.5 * (phi_b[:, :, None] * eye[:, None, :]).reshape(HEADS * HID, HEADS)

    h, sa, sb = pl.pallas_call(
        _proj_body,
        out_shape=[
            jax.ShapeDtypeStruct((N, HEADS * HID), jnp.float32),
            jax.ShapeDtypeStruct((N, HEADS), jnp.float32),
            jax.ShapeDtypeStruct((N, HEADS), jnp.float32),
        ],
    )(x, Wf, pa, pb)

    sbt = sb.T
    bat3 = batch.reshape(NB, 1, BI)
    b1r = b1.reshape(1, HID)
    b2r = b2.reshape(1, NC_OUT)

    grid = (NB,)
    logyp, preds = pl.pallas_call(
        _main_body,
        grid=grid,
        in_specs=[
            pl.BlockSpec((BI, N), lambda i: (i, 0)),
            pl.BlockSpec((N, HEADS * HID), lambda i: (0, 0)),
            pl.BlockSpec((BI, HEADS), lambda i: (i, 0)),
            pl.BlockSpec((HEADS, N), lambda i: (0, 0)),
            pl.BlockSpec((1, 1, BI), lambda i: (i, 0, 0)),
            pl.BlockSpec((HEADS * HID, HID), lambda i: (0, 0)),
            pl.BlockSpec((1, HID), lambda i: (0, 0)),
            pl.BlockSpec((HID, NC_OUT), lambda i: (0, 0)),
            pl.BlockSpec((1, NC_OUT), lambda i: (0, 0)),
        ],
        out_specs=[
            pl.BlockSpec((NG, NC_OUT), lambda i: (0, 0)),
            pl.BlockSpec((BI, NC_OUT), lambda i: (i, 0)),
        ],
        out_shape=[
            jax.ShapeDtypeStruct((NG, NC_OUT), jnp.float32),
            jax.ShapeDtypeStruct((N, NC_OUT), jnp.float32),
        ],
        scratch_shapes=[
            pltpu.VMEM((NG, NC_OUT), jnp.float32),
            pltpu.VMEM((NG, 1), jnp.float32),
        ],
        compiler_params=pltpu.CompilerParams(
            dimension_semantics=("arbitrary",),
        ),
    )(mask, h, sa, sbt, bat3, W1, b1r, W2, b2r)

    return (logyp, preds)


# speedup vs baseline: 2.3810x; 1.1395x over previous
import jax
import jax.numpy as jnp
from jax.experimental import pallas as pl
from jax.experimental.pallas import tpu as pltpu

N = 4096
D_IN = 128
HEADS = 4
HID = 16
NC_OUT = 16
NG = 64
EPS = 0.0001

BI = 256
NB = N // BI

_HIGH = jax.lax.Precision.DEFAULT


def _proj_body(x_ref, wf_ref, pab_ref, h_ref, sab_ref):
    h = jax.lax.dot(x_ref[...], wf_ref[...], precision=_HIGH)
    h_ref[...] = h
    sab_ref[...] = jax.lax.dot(h, pab_ref[...], precision=_HIGH)


def _main_body(mask_ref, h_ref, sa_ref, sbt_ref, bat_ref,
               w1_ref, b1_ref, w2_ref, b2_ref,
               logyp_ref, preds_ref, sums_ref, cnt_ref):
    ib = pl.program_id(0)

    @pl.when(ib == 0)
    def _init():
        sums_ref[...] = jnp.zeros_like(sums_ref)
        cnt_ref[...] = jnp.zeros_like(cnt_ref)

    bf = jnp.bfloat16
    e2 = jnp.where(mask_ref[...] == 1, 0.5, 0.0).astype(bf)
    h_bf = h_ref[...].astype(bf)
    sa_bf = sa_ref[...].astype(bf)
    sbt_bf = sbt_ref[...].astype(bf)
    base = jnp.dot(e2, h_bf, preferred_element_type=jnp.float32)
    aggs = []
    for hd in range(HEADS):
        t = sa_bf[:, hd:hd + 1] + sbt_bf[hd:hd + 1, :]
        m = jnp.maximum(t, bf(0.01) * t)
        v = jnp.tanh(m)
        wmv = e2 * v
        hh = h_bf[:, hd * HID:(hd + 1) * HID]
        aggs.append(jnp.dot(wmv, hh, preferred_element_type=jnp.float32))
    agg = base + jnp.concatenate(aggs, axis=1)

    z = jax.lax.dot(agg, w1_ref[...], precision=_HIGH) + b1_ref[...]
    z = jnp.maximum(z, 0.01 * z)
    z = jax.lax.dot(z, w2_ref[...], precision=_HIGH) + b2_ref[...]
    tmp = jnp.exp(z - jnp.max(z, axis=-1, keepdims=True)) + EPS
    preds = tmp / jnp.sum(tmp, axis=-1, keepdims=True)
    preds_ref[...] = preds

    bat = bat_ref[0]
    gi = jax.lax.broadcasted_iota(jnp.int32, (NG, BI), 0)
    oh = (gi == bat).astype(jnp.float32)
    sums_ref[...] += jax.lax.dot(oh, preds, precision=_HIGH)
    cnt_ref[...] += jnp.sum(oh, axis=1, keepdims=True)

    @pl.when(ib == NB - 1)
    def _fin():
        yp = sums_ref[...] / jnp.maximum(cnt_ref[...], 1.0)
        logyp_ref[...] = jnp.log(yp)


@jax.jit
def kernel(x, batch, mask, Wf, W1, b1, W2, b2, phi):
    phi_a = phi[:, :HID, 0]
    phi_b = phi[:, HID:, 0]
    eye = jnp.eye(HEADS, dtype=jnp.float32)
    pa = 0.5 * (phi_a[:, :, None] * eye[:, None, :]).reshape(HEADS * HID, HEADS)
    pb = 0.5 * (phi_b[:, :, None] * eye[:, None, :]).reshape(HEADS * HID, HEADS)

    pab = jnp.concatenate([pa, pb], axis=1)
    h, sab = pl.pallas_call(
        _proj_body,
        out_shape=[
            jax.ShapeDtypeStruct((N, HEADS * HID), jnp.float32),
            jax.ShapeDtypeStruct((N, 2 * HEADS), jnp.float32),
        ],
    )(x, Wf, pab)

    sa = sab[:, :HEADS]
    sbt = sab[:, HEADS:].T
    bat3 = batch.reshape(NB, 1, BI)
    b1r = b1.reshape(1, HID)
    b2r = b2.reshape(1, NC_OUT)

    grid = (NB,)
    logyp, preds = pl.pallas_call(
        _main_body,
        grid=grid,
        in_specs=[
            pl.BlockSpec((BI, N), lambda i: (i, 0)),
            pl.BlockSpec((N, HEADS * HID), lambda i: (0, 0)),
            pl.BlockSpec((BI, HEADS), lambda i: (i, 0)),
            pl.BlockSpec((HEADS, N), lambda i: (0, 0)),
            pl.BlockSpec((1, 1, BI), lambda i: (i, 0, 0)),
            pl.BlockSpec((HEADS * HID, HID), lambda i: (0, 0)),
            pl.BlockSpec((1, HID), lambda i: (0, 0)),
            pl.BlockSpec((HID, NC_OUT), lambda i: (0, 0)),
            pl.BlockSpec((1, NC_OUT), lambda i: (0, 0)),
        ],
        out_specs=[
            pl.BlockSpec((NG, NC_OUT), lambda i: (0, 0)),
            pl.BlockSpec((BI, NC_OUT), lambda i: (i, 0)),
        ],
        out_shape=[
            jax.ShapeDtypeStruct((NG, NC_OUT), jnp.float32),
            jax.ShapeDtypeStruct((N, NC_OUT), jnp.float32),
        ],
        scratch_shapes=[
            pltpu.VMEM((NG, NC_OUT), jnp.float32),
            pltpu.VMEM((NG, 1), jnp.float32),
        ],
        compiler_params=pltpu.CompilerParams(
            dimension_semantics=("arbitrary",),
        ),
    )(mask, h, sa, sbt, bat3, W1, b1r, W2, b2r)

    return (logyp, preds)


# speedup vs baseline: 2.4583x; 1.0325x over previous
import jax
import jax.numpy as jnp
from jax.experimental import pallas as pl
from jax.experimental.pallas import tpu as pltpu

N = 4096
D_IN = 128
HEADS = 4
HID = 16
NC_OUT = 16
NG = 64
EPS = 0.0001

BI = 256
NB = N // BI

_HIGH = jax.lax.Precision.DEFAULT


def _proj_body(x_ref, wf_ref, pab_ref, h_ref, sab_ref, sabt_ref):
    h = jax.lax.dot(x_ref[...], wf_ref[...], precision=_HIGH)
    h_ref[...] = h
    sab = jax.lax.dot(h, pab_ref[...], precision=_HIGH)
    sab_ref[...] = sab
    sabt_ref[...] = jnp.transpose(sab)


def _main_body(mask_ref, h_ref, sa_ref, sbt_ref, bat_ref,
               w1_ref, b1_ref, w2_ref, b2_ref,
               logyp_ref, preds_ref, sums_ref, cnt_ref):
    ib = pl.program_id(0)

    @pl.when(ib == 0)
    def _init():
        sums_ref[...] = jnp.zeros_like(sums_ref)
        cnt_ref[...] = jnp.zeros_like(cnt_ref)

    bf = jnp.bfloat16
    e2 = jnp.where(mask_ref[...] == 1, 0.5, 0.0).astype(bf)
    h_bf = h_ref[...].astype(bf)
    sa_bf = sa_ref[:, :HEADS].astype(bf)
    sbt_bf = sbt_ref[HEADS:, :].astype(bf)
    base = jnp.dot(e2, h_bf, preferred_element_type=jnp.float32)
    aggs = []
    for hd in range(HEADS):
        t = sa_bf[:, hd:hd + 1] + sbt_bf[hd:hd + 1, :]
        m = jnp.maximum(t, bf(0.01) * t)
        v = jnp.tanh(m)
        wmv = e2 * v
        hh = h_bf[:, hd * HID:(hd + 1) * HID]
        aggs.append(jnp.dot(wmv, hh, preferred_element_type=jnp.float32))
    agg = base + jnp.concatenate(aggs, axis=1)

    z = jax.lax.dot(agg, w1_ref[...], precision=_HIGH) + b1_ref[...]
    z = jnp.maximum(z, 0.01 * z)
    z = jax.lax.dot(z, w2_ref[...], precision=_HIGH) + b2_ref[...]
    tmp = jnp.exp(z - jnp.max(z, axis=-1, keepdims=True)) + EPS
    preds = tmp / jnp.sum(tmp, axis=-1, keepdims=True)
    preds_ref[...] = preds

    bat = bat_ref[0]
    gi = jax.lax.broadcasted_iota(jnp.int32, (NG, BI), 0)
    oh = (gi == bat).astype(jnp.float32)
    sums_ref[...] += jax.lax.dot(oh, preds, precision=_HIGH)
    cnt_ref[...] += jnp.sum(oh, axis=1, keepdims=True)

    @pl.when(ib == NB - 1)
    def _fin():
        yp = sums_ref[...] / jnp.maximum(cnt_ref[...], 1.0)
        logyp_ref[...] = jnp.log(yp)


@jax.jit
def kernel(x, batch, mask, Wf, W1, b1, W2, b2, phi):
    phi_a = phi[:, :HID, 0]
    phi_b = phi[:, HID:, 0]
    eye = jnp.eye(HEADS, dtype=jnp.float32)
    pa = 0.5 * (phi_a[:, :, None] * eye[:, None, :]).reshape(HEADS * HID, HEADS)
    pb = 0.5 * (phi_b[:, :, None] * eye[:, None, :]).reshape(HEADS * HID, HEADS)

    pab = jnp.concatenate([pa, pb], axis=1)
    h, sab, sabt = pl.pallas_call(
        _proj_body,
        out_shape=[
            jax.ShapeDtypeStruct((N, HEADS * HID), jnp.float32),
            jax.ShapeDtypeStruct((N, 2 * HEADS), jnp.float32),
            jax.ShapeDtypeStruct((2 * HEADS, N), jnp.float32),
        ],
    )(x, Wf, pab)
    bat3 = batch.reshape(NB, 1, BI)
    b1r = b1.reshape(1, HID)
    b2r = b2.reshape(1, NC_OUT)

    grid = (NB,)
    logyp, preds = pl.pallas_call(
        _main_body,
        grid=grid,
        in_specs=[
            pl.BlockSpec((BI, N), lambda i: (i, 0)),
            pl.BlockSpec((N, HEADS * HID), lambda i: (0, 0)),
            pl.BlockSpec((BI, 2 * HEADS), lambda i: (i, 0)),
            pl.BlockSpec((2 * HEADS, N), lambda i: (0, 0)),
            pl.BlockSpec((1, 1, BI), lambda i: (i, 0, 0)),
            pl.BlockSpec((HEADS * HID, HID), lambda i: (0, 0)),
            pl.BlockSpec((1, HID), lambda i: (0, 0)),
            pl.BlockSpec((HID, NC_OUT), lambda i: (0, 0)),
            pl.BlockSpec((1, NC_OUT), lambda i: (0, 0)),
        ],
        out_specs=[
            pl.BlockSpec((NG, NC_OUT), lambda i: (0, 0)),
            pl.BlockSpec((BI, NC_OUT), lambda i: (i, 0)),
        ],
        out_shape=[
            jax.ShapeDtypeStruct((NG, NC_OUT), jnp.float32),
            jax.ShapeDtypeStruct((N, NC_OUT), jnp.float32),
        ],
        scratch_shapes=[
            pltpu.VMEM((NG, NC_OUT), jnp.float32),
            pltpu.VMEM((NG, 1), jnp.float32),
        ],
        compiler_params=pltpu.CompilerParams(
            dimension_semantics=("arbitrary",),
        ),
    )(mask, h, sab, sabt, bat3, W1, b1r, W2, b2r)

    return (logyp, preds)


# speedup vs baseline: 2.6067x; 1.0604x over previous
import jax
import jax.numpy as jnp
from jax.experimental import pallas as pl
from jax.experimental.pallas import tpu as pltpu

N = 4096
D_IN = 128
HEADS = 4
HID = 16
NC_OUT = 16
NG = 64
EPS = 0.0001

BI = 256
N2 = N // 2
NB2 = N2 // BI


def _half(mask_blk, row0, ib, bat_row, h_bf, sab_sc, sabt_sc,
          w1_ref, b1_ref, w2_ref, b2_ref, sums_ref, cnt_ref):
    bf = jnp.bfloat16
    e2 = jnp.where(mask_blk == 1, 0.5, 0.0).astype(bf)
    sa_bf = sab_sc[pl.ds(row0 + ib * BI, BI), :HEADS].astype(bf)
    sbt_bf = sabt_sc[HEADS:, :].astype(bf)
    base = jnp.dot(e2, h_bf, preferred_element_type=jnp.float32)
    aggs = []
    for hd in range(HEADS):
        t = sa_bf[:, hd:hd + 1] + sbt_bf[hd:hd + 1, :]
        m = jnp.maximum(t, bf(0.01) * t)
        v = jnp.tanh(m)
        wmv = e2 * v
        hh = h_bf[:, hd * HID:(hd + 1) * HID]
        aggs.append(jnp.dot(wmv, hh, preferred_element_type=jnp.float32))
    agg = base + jnp.concatenate(aggs, axis=1)

    z = jax.lax.dot(agg, w1_ref[...]) + b1_ref[...]
    z = jnp.maximum(z, 0.01 * z)
    z = jax.lax.dot(z, w2_ref[...]) + b2_ref[...]
    tmp = jnp.exp(z - jnp.max(z, axis=-1, keepdims=True)) + EPS
    preds = tmp / jnp.sum(tmp, axis=-1, keepdims=True)

    gi = jax.lax.broadcasted_iota(jnp.int32, (NG, BI), 0)
    oh = (gi == bat_row).astype(jnp.float32)
    sums_ref[...] += jax.lax.dot(oh, preds)
    cnt_ref[...] += jnp.sum(oh, axis=1, keepdims=True)
    return preds


def _main_body(x_ref, wf_ref, pab_ref, mtop_ref, mbot_ref, bat_ref,
               w1_ref, b1_ref, w2_ref, b2_ref,
               logyp_ref, preds_ref,
               h_sc, sab_sc, sabt_sc, sums_ref, cnt_ref):
    ib = pl.program_id(0)

    @pl.when(ib == 0)
    def _init():
        hp = jax.lax.dot(x_ref[...], wf_ref[...])
        h_sc[...] = hp
        sab = jax.lax.dot(hp, pab_ref[...])
        sab_sc[...] = sab
        sabt_sc[...] = jnp.transpose(sab)
        sums_ref[...] = jnp.zeros_like(sums_ref)
        cnt_ref[...] = jnp.zeros_like(cnt_ref)

    h_bf = h_sc[...].astype(jnp.bfloat16)
    preds_ref[0] = _half(mtop_ref[0], 0, ib, bat_ref[0, 0], h_bf,
                         sab_sc, sabt_sc, w1_ref, b1_ref, w2_ref, b2_ref,
                         sums_ref, cnt_ref)
    preds_ref[1] = _half(mbot_ref[0], N2, ib, bat_ref[1, 0], h_bf,
                         sab_sc, sabt_sc, w1_ref, b1_ref, w2_ref, b2_ref,
                         sums_ref, cnt_ref)

    @pl.when(ib == NB2 - 1)
    def _fin():
        yp = sums_ref[...] / jnp.maximum(cnt_ref[...], 1.0)
        logyp_ref[...] = jnp.log(yp)


@jax.jit
def kernel(x, batch, mask, Wf, W1, b1, W2, b2, phi):
    phi_a = phi[:, :HID, 0]
    phi_b = phi[:, HID:, 0]
    eye = jnp.eye(HEADS, dtype=jnp.float32)
    pa = 0.5 * (phi_a[:, :, None] * eye[:, None, :]).reshape(HEADS * HID, HEADS)
    pb = 0.5 * (phi_b[:, :, None] * eye[:, None, :]).reshape(HEADS * HID, HEADS)
    pab = jnp.concatenate([pa, pb], axis=1)

    mask3 = mask.reshape(2, N2, N)
    bat4 = batch.reshape(2, NB2, 1, BI)
    b1r = b1.reshape(1, HID)
    b2r = b2.reshape(1, NC_OUT)

    grid = (NB2,)
    logyp, preds2 = pl.pallas_call(
        _main_body,
        grid=grid,
        in_specs=[
            pl.BlockSpec((N, D_IN), lambda i: (0, 0)),
            pl.BlockSpec((D_IN, HEADS * HID), lambda i: (0, 0)),
            pl.BlockSpec((HEADS * HID, 2 * HEADS), lambda i: (0, 0)),
            pl.BlockSpec((1, BI, N), lambda i: (0, i, 0)),
            pl.BlockSpec((1, BI, N), lambda i: (1, i, 0)),
            pl.BlockSpec((2, 1, 1, BI), lambda i: (0, i, 0, 0)),
            pl.BlockSpec((HEADS * HID, HID), lambda i: (0, 0)),
            pl.BlockSpec((1, HID), lambda i: (0, 0)),
            pl.BlockSpec((HID, NC_OUT), lambda i: (0, 0)),
            pl.BlockSpec((1, NC_OUT), lambda i: (0, 0)),
        ],
        out_specs=[
            pl.BlockSpec((NG, NC_OUT), lambda i: (0, 0)),
            pl.BlockSpec((2, BI, NC_OUT), lambda i: (0, i, 0)),
        ],
        out_shape=[
            jax.ShapeDtypeStruct((NG, NC_OUT), jnp.float32),
            jax.ShapeDtypeStruct((2, N2, NC_OUT), jnp.float32),
        ],
        scratch_shapes=[
            pltpu.VMEM((N, HEADS * HID), jnp.float32),
            pltpu.VMEM((N, 2 * HEADS), jnp.float32),
            pltpu.VMEM((2 * HEADS, N), jnp.float32),
            pltpu.VMEM((NG, NC_OUT), jnp.float32),
            pltpu.VMEM((NG, 1), jnp.float32),
        ],
        compiler_params=pltpu.CompilerParams(
            dimension_semantics=("arbitrary",),
        ),
    )(x, Wf, pab, mask3, mask3, bat4, W1, b1r, W2, b2r)

    return (logyp, preds2.reshape(N, NC_OUT))


# speedup vs baseline: 2.6221x; 1.0059x over previous
import jax
import jax.numpy as jnp
from jax.experimental import pallas as pl
from jax.experimental.pallas import tpu as pltpu

N = 4096
D_IN = 128
HEADS = 4
HID = 16
NC_OUT = 16
NG = 64
EPS = 0.0001

BI = 256
N2 = N // 2
NB2 = N2 // BI


def _half(mask_blk, row0, ib, h_bf, sab_sc, sabt_sc,
          w1_ref, b1_ref, w2_ref, b2_ref):
    bf = jnp.bfloat16
    e2 = jnp.where(mask_blk == 1, 0.5, 0.0).astype(bf)
    sa_bf = sab_sc[pl.ds(row0 + ib * BI, BI), :HEADS]
    sbt_bf = sabt_sc[HEADS:, :]
    base = jnp.dot(e2, h_bf, preferred_element_type=jnp.float32)
    aggs = []
    for hd in range(HEADS):
        t = sa_bf[:, hd:hd + 1] + sbt_bf[hd:hd + 1, :]
        m = jnp.maximum(t, bf(0.01) * t)
        v = jnp.tanh(m)
        wmv = e2 * v
        hh = h_bf[:, hd * HID:(hd + 1) * HID]
        aggs.append(jnp.dot(wmv, hh, preferred_element_type=jnp.float32))
    agg = base + jnp.concatenate(aggs, axis=1)

    z = jax.lax.dot(agg, w1_ref[...]) + b1_ref[...]
    z = jnp.maximum(z, 0.01 * z)
    z = jax.lax.dot(z, w2_ref[...]) + b2_ref[...]
    tmp = jnp.exp(z - jnp.max(z, axis=-1, keepdims=True)) + EPS
    return tmp / jnp.sum(tmp, axis=-1, keepdims=True)


def _main_body(x_ref, wf_ref, pab_ref, mtop_ref, mbot_ref, batw_ref,
               w1_ref, b1_ref, w2_ref, b2_ref,
               logyp_ref, preds_ref,
               h_sc, sab_sc, sabt_sc):
    ib = pl.program_id(0)
    bf = jnp.bfloat16

    @pl.when(ib == 0)
    def _init():
        hp = jax.lax.dot(x_ref[...], wf_ref[...])
        sab = jax.lax.dot(hp, pab_ref[...])
        h_sc[...] = hp.astype(bf)
        sab_sc[...] = sab.astype(bf)
        sabt_sc[...] = jnp.transpose(sab).astype(bf)

    h_bf = h_sc[...]
    preds_ref[pl.ds(ib * BI, BI), :] = _half(
        mtop_ref[0], 0, ib, h_bf, sab_sc, sabt_sc,
        w1_ref, b1_ref, w2_ref, b2_ref)
    preds_ref[pl.ds(N2 + ib * BI, BI), :] = _half(
        mbot_ref[0], N2, ib, h_bf, sab_sc, sabt_sc,
        w1_ref, b1_ref, w2_ref, b2_ref)

    @pl.when(ib == NB2 - 1)
    def _fin():
        bat = batw_ref[...]
        gi = jax.lax.broadcasted_iota(jnp.int32, (NG, N), 0)
        oh = (gi == bat).astype(jnp.float32)
        sums = jax.lax.dot(oh, preds_ref[...])
        cnt = jnp.sum(oh, axis=1, keepdims=True)
        yp = sums / jnp.maximum(cnt, 1.0)
        logyp_ref[...] = jnp.log(yp)


@jax.jit
def kernel(x, batch, mask, Wf, W1, b1, W2, b2, phi):
    phi_a = phi[:, :HID, 0]
    phi_b = phi[:, HID:, 0]
    eye = jnp.eye(HEADS, dtype=jnp.float32)
    pa = 0.5 * (phi_a[:, :, None] * eye[:, None, :]).reshape(HEADS * HID, HEADS)
    pb = 0.5 * (phi_b[:, :, None] * eye[:, None, :]).reshape(HEADS * HID, HEADS)
    pab = jnp.concatenate([pa, pb], axis=1)

    mask3 = mask.reshape(2, N2, N)
    batw = batch.reshape(1, N)
    b1r = b1.reshape(1, HID)
    b2r = b2.reshape(1, NC_OUT)

    grid = (NB2,)
    logyp, preds = pl.pallas_call(
        _main_body,
        grid=grid,
        in_specs=[
            pl.BlockSpec((N, D_IN), lambda i: (0, 0)),
            pl.BlockSpec((D_IN, HEADS * HID), lambda i: (0, 0)),
            pl.BlockSpec((HEADS * HID, 2 * HEADS), lambda i: (0, 0)),
            pl.BlockSpec((1, BI, N), lambda i: (0, i, 0)),
            pl.BlockSpec((1, BI, N), lambda i: (1, i, 0)),
            pl.BlockSpec((1, N), lambda i: (0, 0)),
            pl.BlockSpec((HEADS * HID, HID), lambda i: (0, 0)),
            pl.BlockSpec((1, HID), lambda i: (0, 0)),
            pl.BlockSpec((HID, NC_OUT), lambda i: (0, 0)),
            pl.BlockSpec((1, NC_OUT), lambda i: (0, 0)),
        ],
        out_specs=[
            pl.BlockSpec((NG, NC_OUT), lambda i: (0, 0)),
            pl.BlockSpec((N, NC_OUT), lambda i: (0, 0)),
        ],
        out_shape=[
            jax.ShapeDtypeStruct((NG, NC_OUT), jnp.float32),
            jax.ShapeDtypeStruct((N, NC_OUT), jnp.float32),
        ],
        scratch_shapes=[
            pltpu.VMEM((N, HEADS * HID), jnp.bfloat16),
            pltpu.VMEM((N, 2 * HEADS), jnp.bfloat16),
            pltpu.VMEM((2 * HEADS, N), jnp.bfloat16),
        ],
        compiler_params=pltpu.CompilerParams(
            dimension_semantics=("arbitrary",),
        ),
    )(x, Wf, pab, mask3, mask3, batw, W1, b1r, W2, b2r)

    return (logyp, preds)
